# baseline (speedup 1.0000x reference)
import functools

import jax
import jax.numpy as jnp
from jax import lax
from jax.experimental import pallas as pl
from jax.experimental.pallas import tpu as pltpu
from jax.experimental.pallas import tpu_sc as plsc

_N = 10000
_E = 160000
_DIN = 256
_H = 512
_OUT = 128
_B = 128
_L = 4

_CW = 128
_NCH = _H // _CW
_NS = 16
_RPT = 624
_RTAIL = _N - _RPT * _NS
_EB = 128
_EPT_PAD = 10240
_NB = _EPT_PAD // _EB
_NBUF = 1
_TRASH = _N
_RB = 1000
_GRID = _N // _RB


def _make_seg_sum():
    mesh = plsc.VectorSubcoreMesh(core_axis_name="c", subcore_axis_name="s",
                                  num_cores=2, num_subcores=_NS)

    @functools.partial(
        pl.kernel,
        out_type=[jax.ShapeDtypeStruct((_N, _CW), jnp.float32)] * _NCH,
        mesh=mesh,
        scratch_types=[pltpu.VMEM((_EB,), jnp.int32)] * (2 * _NBUF)
          + [pltpu.VMEM((_EB, _CW), jnp.float32)] * _NBUF
          + [pltpu.VMEM_SHARED((_N + 128, _CW), jnp.float32)]
          + [pltpu.SemaphoreType.DMA] * (2 * _NBUF),
    )
    def seg(h0, h1, h2, h3, src_hbm, dst_hbm, o0, o1, o2, o3, *rest):
        sidx = rest[:_NBUF]
        didx = rest[_NBUF:2 * _NBUF]
        rows = rest[2 * _NBUF:3 * _NBUF]
        acc = rest[3 * _NBUF]
        gsem = rest[3 * _NBUF + 1:3 * _NBUF + 1 + _NBUF]
        ssem = rest[3 * _NBUF + 1 + _NBUF:]
        c = lax.axis_index("c")
        s = lax.axis_index("s")
        row0 = pl.multiple_of(s * _RPT, 8)
        ebase = s * _EPT_PAD

        def do_chunk(h_hbm, out_hbm):
            pltpu.sync_copy(h_hbm.at[pl.ds(row0, _RPT)],
                            acc.at[pl.ds(row0, _RPT)])

            @pl.when(s == _NS - 1)
            def _init_tail():
                pltpu.sync_copy(h_hbm.at[pl.ds(_RPT * _NS, _RTAIL)],
                                acc.at[pl.ds(_RPT * _NS, _RTAIL)])

            plsc.subcore_barrier()

            def body(j, carry):
                gds = []
                for b in range(_NBUF):
                    base = pl.multiple_of(ebase + (j * _NBUF + b) * _EB, 8)
                    pltpu.sync_copy(src_hbm.at[pl.ds(base, _EB)], sidx[b])
                    pltpu.sync_copy(dst_hbm.at[pl.ds(base, _EB)], didx[b])
                    gds.append(pltpu.async_copy(
                        h_hbm.at[sidx[b]], rows[b], gsem[b]))
                for b in range(_NBUF):
                    gds[b].wait()
                    pltpu.sync_copy(rows[b], acc.at[didx[b]], add=True)
                return carry

            lax.fori_loop(0, _NB // _NBUF, body, 0)
            plsc.subcore_barrier()
            pltpu.sync_copy(acc.at[pl.ds(row0, _RPT)],
                            out_hbm.at[pl.ds(row0, _RPT)])

            @pl.when(s == _NS - 1)
            def _wb_tail():
                pltpu.sync_copy(acc.at[pl.ds(_RPT * _NS, _RTAIL)],
                                out_hbm.at[pl.ds(_RPT * _NS, _RTAIL)])

            plsc.subcore_barrier()

        def path01():
            do_chunk(h0, o0)
            do_chunk(h1, o1)

        def path23():
            do_chunk(h2, o2)
            do_chunk(h3, o3)

        lax.cond(c == 0, path01, path23)

    return seg


_seg_sum_cache = None


def _seg_sum(*args):
    global _seg_sum_cache
    if _seg_sum_cache is None:
        _seg_sum_cache = _make_seg_sum()
    return _seg_sum_cache(*args)


def _enc_body(x_ref, w_ref, b_ref, o0, o1, o2, o3):
    y = jnp.dot(x_ref[...], w_ref[...],
                preferred_element_type=jnp.float32) + b_ref[...]
    o0[...] = y[:, 0:128]
    o1[...] = y[:, 128:256]
    o2[...] = y[:, 256:384]
    o3[...] = y[:, 384:512]


_enc_call = pl.pallas_call(
    _enc_body,
    grid=(_GRID,),
    in_specs=[
        pl.BlockSpec((_RB, _DIN), lambda i: (i, 0)),
        pl.BlockSpec((_DIN, _H), lambda i: (0, 0)),
        pl.BlockSpec((1, _H), lambda i: (0, 0)),
    ],
    out_specs=[pl.BlockSpec((_RB, _CW), lambda i: (i, 0))] * _NCH,
    out_shape=[jax.ShapeDtypeStruct((_N, _CW), jnp.float32)] * _NCH,
)


def _mlp_body(z0, z1, z2, z3, w1, b1, w2, b2, o0, o1, o2, o3):
    zin = jnp.concatenate([z0[...], z1[...], z2[...], z3[...]], axis=1)
    y = jnp.dot(zin, w1[...], preferred_element_type=jnp.float32) + b1[...]
    y = jnp.maximum(y, 0.0)
    z = jnp.dot(y, w2[...], preferred_element_type=jnp.float32) + b2[...]
    z = jnp.maximum(z, 0.0)
    o0[...] = z[:, 0:128]
    o1[...] = z[:, 128:256]
    o2[...] = z[:, 256:384]
    o3[...] = z[:, 384:512]


_mlp_call = pl.pallas_call(
    _mlp_body,
    grid=(_GRID,),
    in_specs=[pl.BlockSpec((_RB, _CW), lambda i: (i, 0))] * _NCH + [
        pl.BlockSpec((_H, _H), lambda i: (0, 0)),
        pl.BlockSpec((1, _H), lambda i: (0, 0)),
        pl.BlockSpec((_H, _H), lambda i: (0, 0)),
        pl.BlockSpec((1, _H), lambda i: (0, 0)),
    ],
    out_specs=[pl.BlockSpec((_RB, _CW), lambda i: (i, 0))] * _NCH,
    out_shape=[jax.ShapeDtypeStruct((_N, _CW), jnp.float32)] * _NCH,
)


def _final_body(*refs):
    zs = refs[:_L * _NCH]
    (wj, bj, batch3, gf, wg, bg, wd1a, wd1b, bd1, wd2, bd2) = \
        refs[_L * _NCH:_L * _NCH + 11]
    out = refs[-3]
    pool_acc = refs[-2]
    cnt_acc = refs[-1]
    i = pl.program_id(0)

    @pl.when(i == 0)
    def _init():
        pool_acc[...] = jnp.zeros((_B, _H), jnp.float32)
        cnt_acc[...] = jnp.zeros((_B, _B), jnp.float32)

    wjv = wj[...]
    jb = bj[...]
    for k in range(_L * _NCH):
        jb = jb + jnp.dot(zs[k][...], wjv[k * _CW:(k + 1) * _CW, :],
                          preferred_element_type=jnp.float32)

    bb = batch3[0, 0, :]
    oh = (bb[:, None] == lax.broadcasted_iota(jnp.int32, (1, _B), 1)
          ).astype(jnp.float32)
    pool_acc[...] += lax.dot_general(
        oh, jb, (((0,), (0,)), ((), ())),
        preferred_element_type=jnp.float32)
    cnt_acc[...] += lax.dot_general(
        oh, jnp.ones((_RB, _B), jnp.float32), (((0,), (0,)), ((), ())),
        preferred_element_type=jnp.float32)

    @pl.when(i == _GRID - 1)
    def _decode():
        counts = jnp.maximum(cnt_acc[...][:, 0:1], 1.0)
        pooled = pool_acc[...] / counts
        ge = jnp.maximum(
            jnp.dot(gf[...], wg[...], preferred_element_type=jnp.float32)
            + bg[...], 0.0)
        d = (jnp.dot(pooled, wd1a[...], preferred_element_type=jnp.float32)
             + jnp.dot(ge, wd1b[...], preferred_element_type=jnp.float32)
             + bd1[...])
        d = jnp.maximum(d, 0.0)
        logits = jnp.dot(d, wd2[...],
                         preferred_element_type=jnp.float32) + bd2[...]
        m = jnp.max(logits, axis=1, keepdims=True)
        e = jnp.exp(logits - m)
        out[...] = e / jnp.sum(e, axis=1, keepdims=True)


_final_call = pl.pallas_call(
    _final_body,
    grid=(_GRID,),
    in_specs=[pl.BlockSpec((_RB, _CW), lambda i: (i, 0))] * (_L * _NCH) + [
        pl.BlockSpec((_L * _H, _H), lambda i: (0, 0)),
        pl.BlockSpec((1, _H), lambda i: (0, 0)),
        pl.BlockSpec((1, 1, _RB), lambda i: (i, 0, 0)),
        pl.BlockSpec((_B, 4), lambda i: (0, 0)),
        pl.BlockSpec((4, _H), lambda i: (0, 0)),
        pl.BlockSpec((1, _H), lambda i: (0, 0)),
        pl.BlockSpec((_H, _H), lambda i: (0, 0)),
        pl.BlockSpec((_H, _H), lambda i: (0, 0)),
        pl.BlockSpec((1, _H), lambda i: (0, 0)),
        pl.BlockSpec((_H, _OUT), lambda i: (0, 0)),
        pl.BlockSpec((1, _OUT), lambda i: (0, 0)),
    ],
    out_specs=pl.BlockSpec((_B, _OUT), lambda i: (0, 0)),
    out_shape=jax.ShapeDtypeStruct((_B, _OUT), jnp.float32),
    scratch_shapes=[
        pltpu.VMEM((_B, _H), jnp.float32),
        pltpu.VMEM((_B, _B), jnp.float32),
    ],
)

_BN_S = (1.0 + 1e-5) ** -0.5


def _fold(lin, bn):
    g = bn["g"] * _BN_S
    return lin["W"] * g[None, :], (lin["b"] * g + bn["b"])[None, :]


def kernel(x, edge_index, batch, global_features, params):
    src, dst = edge_index[0], edge_index[1]
    pad = _NS * _EPT_PAD - _E
    src_p = jnp.pad(src, (0, pad))
    pad_dst = _TRASH + (jnp.arange(pad, dtype=jnp.int32) % 128)
    dst_p = jnp.concatenate([dst, pad_dst])

    h = _enc_call(x, params["enc"]["W"], params["enc"]["b"][None, :])
    zs = []
    for i in range(_L):
        c = params["convs"][i]
        w1, b1 = _fold(c["lin1"], c["bn1"])
        w2, b2 = _fold(c["lin2"], params["bns"][i])
        a = _seg_sum(h[0], h[1], h[2], h[3], src_p, dst_p)
        h = _mlp_call(a[0], a[1], a[2], a[3], w1, b1, w2, b2)
        zs.extend(h)

    wd1, bd1 = _fold(params["dec1"], params["dec_bn"])
    out = _final_call(
        *zs,
        params["jump"]["W"], params["jump"]["b"][None, :],
        batch.reshape(_GRID, 1, _RB),
        global_features,
        params["glob"]["W"], params["glob"]["b"][None, :],
        wd1[:_H, :], wd1[_H:, :], bd1,
        params["dec2"]["W"], params["dec2"]["b"][None, :],
    )
    return out


# speedup vs baseline: 1.3428x; 1.3428x over previous
import functools

import jax
import jax.numpy as jnp
from jax import lax
from jax.experimental import pallas as pl
from jax.experimental.pallas import tpu as pltpu
from jax.experimental.pallas import tpu_sc as plsc

_N = 10000
_E = 160000
_DIN = 256
_H = 512
_OUT = 128
_B = 128
_L = 4

_CW = 128
_NCH = _H // _CW
_NS = 16
_RPT = 624
_RTAIL = _N - _RPT * _NS
_EB = 128
_EPT_PAD = 10112
_NB = _EPT_PAD // _EB
_TRASH = _N
_RB = 1000
_GRID = _N // _RB


def _make_seg_sum():
    mesh = plsc.VectorSubcoreMesh(core_axis_name="c", subcore_axis_name="s",
                                  num_cores=2, num_subcores=_NS)

    @functools.partial(
        pl.kernel,
        out_type=[jax.ShapeDtypeStruct((_N, _CW), jnp.float32)] * _NCH,
        mesh=mesh,
        scratch_types=[
            pltpu.VMEM((_EB,), jnp.int32),
            pltpu.VMEM((_EB,), jnp.int32),
            pltpu.VMEM((_EB, _CW), jnp.float32),
            pltpu.VMEM_SHARED((_N + 16, _CW), jnp.float32),
            pltpu.SemaphoreType.DMA,
        ],
    )
    def seg(h0, h1, h2, h3, src_hbm, dst_hbm, o0, o1, o2, o3,
            sidx, didx, rows, acc, sem):
        c = lax.axis_index("c")
        s = lax.axis_index("s")
        row0 = pl.multiple_of(s * _RPT, 8)
        ebase = s * _EPT_PAD

        def do_chunk(h_hbm, out_hbm):
            pltpu.sync_copy(h_hbm.at[pl.ds(row0, _RPT)],
                            acc.at[pl.ds(row0, _RPT)])

            @pl.when(s == _NS - 1)
            def _init_tail():
                pltpu.sync_copy(h_hbm.at[pl.ds(_RPT * _NS, _RTAIL)],
                                acc.at[pl.ds(_RPT * _NS, _RTAIL)])

            plsc.subcore_barrier()

            def body(i, carry):
                base = pl.multiple_of(ebase + i * _EB, 8)
                pltpu.sync_copy(src_hbm.at[pl.ds(base, _EB)], sidx)
                pltpu.sync_copy(dst_hbm.at[pl.ds(base, _EB)], didx)
                pltpu.async_copy(h_hbm.at[sidx], rows, sem).wait()
                pltpu.sync_copy(rows, acc.at[didx], add=True)
                return carry

            lax.fori_loop(0, _NB, body, 0)
            plsc.subcore_barrier()
            pltpu.sync_copy(acc.at[pl.ds(row0, _RPT)],
                            out_hbm.at[pl.ds(row0, _RPT)])

            @pl.when(s == _NS - 1)
            def _wb_tail():
                pltpu.sync_copy(acc.at[pl.ds(_RPT * _NS, _RTAIL)],
                                out_hbm.at[pl.ds(_RPT * _NS, _RTAIL)])

            plsc.subcore_barrier()

        def path01():
            do_chunk(h0, o0)
            do_chunk(h1, o1)

        def path23():
            do_chunk(h2, o2)
            do_chunk(h3, o3)

        lax.cond(c == 0, path01, path23)

    return seg


_seg_sum_cache = None


def _seg_sum(*args):
    global _seg_sum_cache
    if _seg_sum_cache is None:
        _seg_sum_cache = _make_seg_sum()
    return _seg_sum_cache(*args)


def _enc_body(x_ref, w_ref, b_ref, o0, o1, o2, o3):
    y = jnp.dot(x_ref[...], w_ref[...],
                preferred_element_type=jnp.float32) + b_ref[...]
    o0[...] = y[:, 0:128]
    o1[...] = y[:, 128:256]
    o2[...] = y[:, 256:384]
    o3[...] = y[:, 384:512]


_enc_call = pl.pallas_call(
    _enc_body,
    grid=(_GRID,),
    in_specs=[
        pl.BlockSpec((_RB, _DIN), lambda i: (i, 0)),
        pl.BlockSpec((_DIN, _H), lambda i: (0, 0)),
        pl.BlockSpec((1, _H), lambda i: (0, 0)),
    ],
    out_specs=[pl.BlockSpec((_RB, _CW), lambda i: (i, 0))] * _NCH,
    out_shape=[jax.ShapeDtypeStruct((_N, _CW), jnp.float32)] * _NCH,
)


def _mlp_body(z0, z1, z2, z3, w1, b1, w2, b2, o0, o1, o2, o3):
    zin = jnp.concatenate([z0[...], z1[...], z2[...], z3[...]], axis=1)
    y = jnp.dot(zin, w1[...], preferred_element_type=jnp.float32) + b1[...]
    y = jnp.maximum(y, 0.0)
    z = jnp.dot(y, w2[...], preferred_element_type=jnp.float32) + b2[...]
    z = jnp.maximum(z, 0.0)
    o0[...] = z[:, 0:128]
    o1[...] = z[:, 128:256]
    o2[...] = z[:, 256:384]
    o3[...] = z[:, 384:512]


_mlp_call = pl.pallas_call(
    _mlp_body,
    grid=(_GRID,),
    in_specs=[pl.BlockSpec((_RB, _CW), lambda i: (i, 0))] * _NCH + [
        pl.BlockSpec((_H, _H), lambda i: (0, 0)),
        pl.BlockSpec((1, _H), lambda i: (0, 0)),
        pl.BlockSpec((_H, _H), lambda i: (0, 0)),
        pl.BlockSpec((1, _H), lambda i: (0, 0)),
    ],
    out_specs=[pl.BlockSpec((_RB, _CW), lambda i: (i, 0))] * _NCH,
    out_shape=[jax.ShapeDtypeStruct((_N, _CW), jnp.float32)] * _NCH,
)


def _final_body(*refs):
    zs = refs[:_L * _NCH]
    (wj, bj, batch3, gf, wg, bg, wd1a, wd1b, bd1, wd2, bd2) = \
        refs[_L * _NCH:_L * _NCH + 11]
    out = refs[-3]
    pool_acc = refs[-2]
    cnt_acc = refs[-1]
    i = pl.program_id(0)

    @pl.when(i == 0)
    def _init():
        pool_acc[...] = jnp.zeros((_B, _H), jnp.float32)
        cnt_acc[...] = jnp.zeros((_B, _B), jnp.float32)

    wjv = wj[...]
    jb = bj[...]
    for k in range(_L * _NCH):
        jb = jb + jnp.dot(zs[k][...], wjv[k * _CW:(k + 1) * _CW, :],
                          preferred_element_type=jnp.float32)

    bb = batch3[0, 0, :]
    oh = (bb[:, None] == lax.broadcasted_iota(jnp.int32, (1, _B), 1)
          ).astype(jnp.float32)
    pool_acc[...] += lax.dot_general(
        oh, jb, (((0,), (0,)), ((), ())),
        preferred_element_type=jnp.float32)
    cnt_acc[...] += lax.dot_general(
        oh, jnp.ones((_RB, _B), jnp.float32), (((0,), (0,)), ((), ())),
        preferred_element_type=jnp.float32)

    @pl.when(i == _GRID - 1)
    def _decode():
        counts = jnp.maximum(cnt_acc[...][:, 0:1], 1.0)
        pooled = pool_acc[...] / counts
        ge = jnp.maximum(
            jnp.dot(gf[...], wg[...], preferred_element_type=jnp.float32)
            + bg[...], 0.0)
        d = (jnp.dot(pooled, wd1a[...], preferred_element_type=jnp.float32)
             + jnp.dot(ge, wd1b[...], preferred_element_type=jnp.float32)
             + bd1[...])
        d = jnp.maximum(d, 0.0)
        logits = jnp.dot(d, wd2[...],
                         preferred_element_type=jnp.float32) + bd2[...]
        m = jnp.max(logits, axis=1, keepdims=True)
        e = jnp.exp(logits - m)
        out[...] = e / jnp.sum(e, axis=1, keepdims=True)


_final_call = pl.pallas_call(
    _final_body,
    grid=(_GRID,),
    in_specs=[pl.BlockSpec((_RB, _CW), lambda i: (i, 0))] * (_L * _NCH) + [
        pl.BlockSpec((_L * _H, _H), lambda i: (0, 0)),
        pl.BlockSpec((1, _H), lambda i: (0, 0)),
        pl.BlockSpec((1, 1, _RB), lambda i: (i, 0, 0)),
        pl.BlockSpec((_B, 4), lambda i: (0, 0)),
        pl.BlockSpec((4, _H), lambda i: (0, 0)),
        pl.BlockSpec((1, _H), lambda i: (0, 0)),
        pl.BlockSpec((_H, _H), lambda i: (0, 0)),
        pl.BlockSpec((_H, _H), lambda i: (0, 0)),
        pl.BlockSpec((1, _H), lambda i: (0, 0)),
        pl.BlockSpec((_H, _OUT), lambda i: (0, 0)),
        pl.BlockSpec((1, _OUT), lambda i: (0, 0)),
    ],
    out_specs=pl.BlockSpec((_B, _OUT), lambda i: (0, 0)),
    out_shape=jax.ShapeDtypeStruct((_B, _OUT), jnp.float32),
    scratch_shapes=[
        pltpu.VMEM((_B, _H), jnp.float32),
        pltpu.VMEM((_B, _B), jnp.float32),
    ],
)

_BN_S = (1.0 + 1e-5) ** -0.5


def _fold(lin, bn):
    g = bn["g"] * _BN_S
    return lin["W"] * g[None, :], (lin["b"] * g + bn["b"])[None, :]


def kernel(x, edge_index, batch, global_features, params):
    src, dst = edge_index[0], edge_index[1]
    src_p = jnp.pad(src.reshape(_NS, _E // _NS),
                    ((0, 0), (0, _EPT_PAD - _E // _NS))).reshape(-1)
    dst_p = jnp.pad(dst.reshape(_NS, _E // _NS),
                    ((0, 0), (0, _EPT_PAD - _E // _NS)),
                    constant_values=_TRASH).reshape(-1)

    h = _enc_call(x, params["enc"]["W"], params["enc"]["b"][None, :])
    zs = []
    for i in range(_L):
        c = params["convs"][i]
        w1, b1 = _fold(c["lin1"], c["bn1"])
        w2, b2 = _fold(c["lin2"], params["bns"][i])
        a = _seg_sum(h[0], h[1], h[2], h[3], src_p, dst_p)
        h = _mlp_call(a[0], a[1], a[2], a[3], w1, b1, w2, b2)
        zs.extend(h)

    wd1, bd1 = _fold(params["dec1"], params["dec_bn"])
    out = _final_call(
        *zs,
        params["jump"]["W"], params["jump"]["b"][None, :],
        batch.reshape(_GRID, 1, _RB),
        global_features,
        params["glob"]["W"], params["glob"]["b"][None, :],
        wd1[:_H, :], wd1[_H:, :], bd1,
        params["dec2"]["W"], params["dec2"]["b"][None, :],
    )
    return out
